# Initial kernel scaffold; baseline (speedup 1.0000x reference)
import jax
import jax.numpy as jnp
from jax.experimental import pallas as pl


def kernel(x, edge_index, edge_weight, W1, b1, W2, b2, Ws, bs):
    raise NotImplementedError("write your pallas kernel here")


# speedup vs baseline: 2.2910x; 2.2910x over previous
import functools

import jax
import jax.numpy as jnp
from jax import lax
from jax.experimental import pallas as pl
from jax.experimental.pallas import tpu as pltpu

_N = 10000
_E = 320000
_F = 128
_H = 128
_K = 64
_EPS = 1e-15
_SELU_SCALE = 1.0507009873554805
_SELU_ALPHA = 1.6732632423543772


def _selu(v):
    return _SELU_SCALE * jnp.where(v > 0, v, _SELU_ALPHA * (jnp.exp(v) - 1.0))


def _mm(a, b):
    return lax.dot_general(a, b, (((1,), (0,)), ((), ())),
                           preferred_element_type=jnp.float32,
                           precision=lax.Precision.DEFAULT)


def _ct(a, b):
    return lax.dot_general(a, b, (((0,), (0,)), ((), ())),
                           preferred_element_type=jnp.float32,
                           precision=lax.Precision.DEFAULT)


def _stage_a_body(x_ref, w1_ref, h1_ref):
    h1_ref[...] = _mm(x_ref[...], w1_ref[...])


def _stage_b_body(h1_ref, degp_ref, table1_ref, dinv_ref):
    deg = degp_ref[...] + 1.0
    dinv = lax.rsqrt(jnp.maximum(deg, 1e-12))
    dinv_ref[...] = dinv
    table1_ref[...] = h1_ref[...] * dinv


def _stage_c_body(p0_ref, p1_ref, table1_ref, dinv_ref, b1_ref, w2_ref,
                  table2_ref):
    dinv = dinv_ref[...]
    agg = dinv * (p0_ref[...] + p1_ref[...] + table1_ref[...]) + b1_ref[...]
    z = _selu(agg)
    table2_ref[...] = _mm(z, w2_ref[...]) * dinv


def _stage_d_body(p0_ref, p1_ref, table2_ref, dinv_ref, b2_ref, ws_ref,
                  bs_ref, s_ref, hfin_ref):
    dinv = dinv_ref[...]
    agg = dinv * (p0_ref[...] + p1_ref[...] + table2_ref[...]) + b2_ref[...]
    z = _selu(agg)
    hfin_ref[...] = z
    logits = _mm(z, ws_ref[...]) + bs_ref[...]
    m = jnp.max(logits, axis=-1, keepdims=True)
    e = jnp.exp(logits - m)
    sm = e / jnp.sum(e, axis=-1, keepdims=True)
    s_ref[...] = sm / jnp.maximum(jnp.sum(sm, axis=-1, keepdims=True), _EPS)


def _stage_f_body(s_ref, hfin_ref, a0_ref, a1_ref, degp_ref,
                  out_ref, oa_ref, sl_ref, ol_ref, cl_ref):
    s = s_ref[...]
    hfin = hfin_ref[...]
    As = a0_ref[...] + a1_ref[...]
    degp = degp_ref[...]

    out_ref[...] = _selu(_ct(s, hfin))
    out_adj = _ct(s, As)
    ca = _ct(s, degp)
    two_m = jnp.maximum(jnp.sum(degp), _EPS)

    eye = (lax.broadcasted_iota(jnp.int32, (_K, _K), 0)
           == lax.broadcasted_iota(jnp.int32, (_K, _K), 1)).astype(jnp.float32)

    tr_adj = jnp.sum(out_adj * eye)
    tr_norm = jnp.sum(ca * ca) / two_m
    sl_ref[...] = jnp.broadcast_to(-(tr_adj - tr_norm) / two_m, (1, 1))

    ss = _ct(s, s)
    nrm_ss = jnp.sqrt(jnp.sum(ss * ss))
    i_nrm = jnp.sqrt(jnp.float32(_K))
    diff = ss / nrm_ss - eye / i_nrm
    ol_ref[...] = jnp.broadcast_to(jnp.sqrt(jnp.sum(diff * diff)), (1, 1))

    cs = jnp.sum(s, axis=0, keepdims=True)
    cl_ref[...] = jnp.broadcast_to(
        jnp.sqrt(jnp.sum(cs * cs)) / _N * jnp.sqrt(jnp.float32(_K)) - 1.0,
        (1, 1))

    m = out_adj * (1.0 - eye)
    d = jnp.sqrt(jnp.maximum(jnp.sum(m, axis=1, keepdims=True), 0.0)) + _EPS
    oa_ref[...] = m / d / jnp.transpose(d)


def _tc_call(body, out_shapes, *args):
    return pl.pallas_call(
        body,
        out_shape=out_shapes,
    )(*args)


def _edge_agg(table, src, dst, ew):
    return jax.ops.segment_sum(ew[:, None] * table[src], dst, num_segments=_N)


def kernel(x, edge_index, edge_weight, W1, b1, W2, b2, Ws, bs):
    src, dst = edge_index[0], edge_index[1]
    ew = edge_weight

    zeros_h = jnp.zeros((_N, _H), jnp.float32)
    zeros_k = jnp.zeros((_N, _K), jnp.float32)
    b1r = b1.reshape(1, _H)
    b2r = b2.reshape(1, _H)
    bsr = bs.reshape(1, _K)

    h1 = _tc_call(_stage_a_body, jax.ShapeDtypeStruct((_N, _H), jnp.float32),
                  x, W1)

    degp = jax.ops.segment_sum(ew, dst, num_segments=_N).reshape(_N, 1)

    table1, dinv = _tc_call(
        _stage_b_body,
        (jax.ShapeDtypeStruct((_N, _H), jnp.float32),
         jax.ShapeDtypeStruct((_N, 1), jnp.float32)),
        h1, degp)

    agg1 = _edge_agg(table1, src, dst, ew)
    table2 = _tc_call(_stage_c_body,
                      jax.ShapeDtypeStruct((_N, _H), jnp.float32),
                      agg1, zeros_h, table1, dinv, b1r, W2)

    agg2 = _edge_agg(table2, src, dst, ew)
    s, hfin = _tc_call(
        _stage_d_body,
        (jax.ShapeDtypeStruct((_N, _K), jnp.float32),
         jax.ShapeDtypeStruct((_N, _H), jnp.float32)),
        agg2, zeros_h, table2, dinv, b2r, Ws, bsr)

    As = _edge_agg(s, src, dst, ew)
    out, oa, sl, ol, cl = _tc_call(
        _stage_f_body,
        (jax.ShapeDtypeStruct((_K, _H), jnp.float32),
         jax.ShapeDtypeStruct((_K, _K), jnp.float32),
         jax.ShapeDtypeStruct((1, 1), jnp.float32),
         jax.ShapeDtypeStruct((1, 1), jnp.float32),
         jax.ShapeDtypeStruct((1, 1), jnp.float32)),
        s, hfin, As, zeros_k, degp)

    return (s, out, oa, sl[0, 0], ol[0, 0], cl[0, 0])


# speedup vs baseline: 4.4338x; 1.9353x over previous
import dataclasses
import functools

import jax
import jax.numpy as jnp
from jax import lax
from jax.experimental import pallas as pl
from jax.experimental.pallas import tpu as pltpu
from jax.experimental.pallas import tpu_sc as plsc

_N = 10000
_E = 320000
_F = 128
_H = 128
_K = 64
_EPS = 1e-15
_SELU_SCALE = 1.0507009873554805
_SELU_ALPHA = 1.6732632423543772


def _selu(v):
    return _SELU_SCALE * jnp.where(v > 0, v, _SELU_ALPHA * (jnp.exp(v) - 1.0))


def _mm(a, b):
    return lax.dot_general(a, b, (((1,), (0,)), ((), ())),
                           preferred_element_type=jnp.float32,
                           precision=lax.Precision.DEFAULT)


def _ct(a, b):
    return lax.dot_general(a, b, (((0,), (0,)), ((), ())),
                           preferred_element_type=jnp.float32,
                           precision=lax.Precision.DEFAULT)


def _stage_a_body(x_ref, w1_ref, h1_ref):
    h1_ref[...] = _mm(x_ref[...], w1_ref[...])


def _stage_b_body(h1_ref, dp_ref, table1_ref, dinv_ref, degp_ref):
    degp = dp_ref[0, :, 0:1] + dp_ref[1, :, 0:1]
    degp_ref[...] = degp
    deg = degp + 1.0
    dinv = lax.rsqrt(jnp.maximum(deg, 1e-12))
    dinv_ref[...] = dinv
    table1_ref[...] = h1_ref[...] * dinv


def _stage_c_body(p_ref, table1_ref, dinv_ref, b1_ref, w2_ref,
                  table2_ref):
    dinv = dinv_ref[...]
    agg = dinv * (p_ref[0] + p_ref[1] + table1_ref[...]) + b1_ref[...]
    z = _selu(agg)
    table2_ref[...] = _mm(z, w2_ref[...]) * dinv


def _stage_d_body(p_ref, table2_ref, dinv_ref, b2_ref, ws_ref,
                  bs_ref, s_ref, hfin_ref):
    dinv = dinv_ref[...]
    agg = dinv * (p_ref[0] + p_ref[1] + table2_ref[...]) + b2_ref[...]
    z = _selu(agg)
    hfin_ref[...] = z
    logits = _mm(z, ws_ref[...]) + bs_ref[...]
    m = jnp.max(logits, axis=-1, keepdims=True)
    e = jnp.exp(logits - m)
    sm = e / jnp.sum(e, axis=-1, keepdims=True)
    s = sm / jnp.maximum(jnp.sum(sm, axis=-1, keepdims=True), _EPS)
    s_ref[:, 0:_K] = s
    s_ref[:, _K:2 * _K] = jnp.zeros((_N, _K), jnp.float32)


def _stage_f_body(s_ref, hfin_ref, a_ref, degp_ref,
                  out_ref, oa_ref, sl_ref, ol_ref, cl_ref):
    s = s_ref[:, 0:_K]
    hfin = hfin_ref[...]
    As = a_ref[0, :, 0:_K] + a_ref[1, :, 0:_K]
    degp = degp_ref[...]

    out_ref[...] = _selu(_ct(s, hfin))
    out_adj = _ct(s, As)
    ca = _ct(s, degp)
    two_m = jnp.maximum(jnp.sum(degp), _EPS)

    eye = (lax.broadcasted_iota(jnp.int32, (_K, _K), 0)
           == lax.broadcasted_iota(jnp.int32, (_K, _K), 1)).astype(jnp.float32)

    tr_adj = jnp.sum(out_adj * eye)
    tr_norm = jnp.sum(ca * ca) / two_m
    sl_ref[...] = jnp.broadcast_to(-(tr_adj - tr_norm) / two_m, (1, 1))

    ss = _ct(s, s)
    nrm_ss = jnp.sqrt(jnp.sum(ss * ss))
    i_nrm = jnp.sqrt(jnp.float32(_K))
    diff = ss / nrm_ss - eye / i_nrm
    ol_ref[...] = jnp.broadcast_to(jnp.sqrt(jnp.sum(diff * diff)), (1, 1))

    cs = jnp.sum(s, axis=0, keepdims=True)
    cl_ref[...] = jnp.broadcast_to(
        jnp.sqrt(jnp.sum(cs * cs)) / _N * jnp.sqrt(jnp.float32(_K)) - 1.0,
        (1, 1))

    m = out_adj * (1.0 - eye)
    d = jnp.sqrt(jnp.maximum(jnp.sum(m, axis=1, keepdims=True), 0.0)) + _EPS
    oa_ref[...] = m / d / jnp.transpose(d)


def _tc_call(body, out_shapes, *args):
    return pl.pallas_call(
        body,
        out_shape=out_shapes,
    )(*args)


_NC = 2
_NS = 16
_NW = _NC * _NS
_EPW = _E // _NW
_BE = 80
_NB = _EPW // _BE
_OSTEP = 624
_OLEN = 640
_ZCH = 128


def _sc_compiler_params():
    cp = pltpu.CompilerParams()
    if "needs_layout_passes" in pltpu.CompilerParams.__dataclass_fields__:
        cp = dataclasses.replace(cp, needs_layout_passes=False)
    return cp


def _sc_edge_agg(table, src, dst, ew, D):
    mesh = plsc.VectorSubcoreMesh(core_axis_name="c", subcore_axis_name="s")

    @functools.partial(
        pl.kernel, mesh=mesh,
        compiler_params=_sc_compiler_params(),
        out_type=jax.ShapeDtypeStruct((_NC, _N, D), jnp.float32),
        scratch_types=[
            pltpu.VMEM((_BE,), jnp.int32),
            pltpu.VMEM((_BE,), jnp.int32),
            pltpu.VMEM((_BE,), jnp.float32),
            pltpu.VMEM((_BE, 128), jnp.float32),
            pltpu.VMEM((_BE, D), jnp.float32),
            pltpu.VMEM((_ZCH, D), jnp.float32),
            pltpu.VMEM_SHARED((_N, D), jnp.float32),
            pltpu.SemaphoreType.DMA,
        ])
    def k(table_hbm, src_hbm, dst_hbm, ew_hbm, out_hbm,
          sidx, didx, ewv, grows, rows, zbuf, acc, sem):
        cid = lax.axis_index("c")
        sid = lax.axis_index("s")
        w = sid * _NC + cid
        r0 = sid * _OSTEP

        @pl.loop(0, _ZCH)
        def _(i):
            for j in range(D // 16):
                zbuf[i, pl.ds(j * 16, 16)] = jnp.zeros((16,), jnp.float32)

        @pl.loop(0, _OLEN // _ZCH)
        def _(cz):
            pltpu.sync_copy(zbuf, acc.at[pl.ds(r0 + cz * _ZCH, _ZCH)])

        plsc.subcore_barrier()

        base0 = w * _EPW

        @pl.loop(0, _NB)
        def _(b):
            base = base0 + b * _BE
            pltpu.sync_copy(src_hbm.at[pl.ds(base, _BE)], sidx)
            pltpu.sync_copy(ew_hbm.at[pl.ds(base, _BE)], ewv)
            pltpu.sync_copy(dst_hbm.at[pl.ds(base, _BE)], didx)
            pltpu.async_copy(table_hbm.at[sidx], grows, sem).wait()

            @pl.loop(0, _BE)
            def _(i):
                wv = plsc.load_gather(ewv, [jnp.full((16,), i, jnp.int32)])
                for j in range(D // 16):
                    rows[i, pl.ds(j * 16, 16)] = \
                        grows[i, pl.ds(j * 16, 16)] * wv

            pltpu.sync_copy(rows, acc.at[didx], add=True)

        plsc.subcore_barrier()
        pltpu.sync_copy(acc.at[pl.ds(r0, _OLEN)],
                        out_hbm.at[cid, pl.ds(r0, _OLEN)])

    return k(table, src, dst, ew)


def _sc_deg(src, dst, ew):
    mesh = plsc.VectorSubcoreMesh(core_axis_name="c", subcore_axis_name="s")

    @functools.partial(
        pl.kernel, mesh=mesh,
        compiler_params=_sc_compiler_params(),
        out_type=jax.ShapeDtypeStruct((_NC, _N, 128), jnp.float32),
        scratch_types=[
            pltpu.VMEM((_BE,), jnp.int32),
            pltpu.VMEM((_BE,), jnp.float32),
            pltpu.VMEM((_BE, 128), jnp.float32),
            pltpu.VMEM((_ZCH, 128), jnp.float32),
            pltpu.VMEM_SHARED((_N, 128), jnp.float32),
        ])
    def k(src_hbm, dst_hbm, ew_hbm, out_hbm, didx, ewv, rows, zbuf, acc):
        cid = lax.axis_index("c")
        sid = lax.axis_index("s")
        w = sid * _NC + cid
        r0 = sid * _OSTEP

        @pl.loop(0, _ZCH)
        def _(i):
            for j in range(8):
                zbuf[i, pl.ds(j * 16, 16)] = jnp.zeros((16,), jnp.float32)

        @pl.loop(0, _OLEN // _ZCH)
        def _(cz):
            pltpu.sync_copy(zbuf, acc.at[pl.ds(r0 + cz * _ZCH, _ZCH)])

        plsc.subcore_barrier()

        base0 = w * _EPW

        @pl.loop(0, _NB)
        def _(b):
            base = base0 + b * _BE
            pltpu.sync_copy(ew_hbm.at[pl.ds(base, _BE)], ewv)
            pltpu.sync_copy(dst_hbm.at[pl.ds(base, _BE)], didx)

            @pl.loop(0, _BE)
            def _(i):
                wv = plsc.load_gather(ewv, [jnp.full((16,), i, jnp.int32)])
                for j in range(8):
                    rows[i, pl.ds(j * 16, 16)] = wv

            pltpu.sync_copy(rows, acc.at[didx], add=True)

        plsc.subcore_barrier()
        pltpu.sync_copy(acc.at[pl.ds(r0, _OLEN)],
                        out_hbm.at[cid, pl.ds(r0, _OLEN)])

    return k(src, dst, ew)


def kernel(x, edge_index, edge_weight, W1, b1, W2, b2, Ws, bs):
    src, dst = edge_index[0], edge_index[1]
    ew = edge_weight
    b1r = b1.reshape(1, _H)
    b2r = b2.reshape(1, _H)
    bsr = bs.reshape(1, _K)

    h1 = _tc_call(_stage_a_body, jax.ShapeDtypeStruct((_N, _H), jnp.float32),
                  x, W1)
    degP = _sc_deg(src, dst, ew)

    table1, dinv, degp = _tc_call(
        _stage_b_body,
        (jax.ShapeDtypeStruct((_N, _H), jnp.float32),
         jax.ShapeDtypeStruct((_N, 1), jnp.float32),
         jax.ShapeDtypeStruct((_N, 1), jnp.float32)),
        h1, degP)

    t1 = _sc_edge_agg(table1, src, dst, ew, _H)
    table2 = _tc_call(_stage_c_body,
                      jax.ShapeDtypeStruct((_N, _H), jnp.float32),
                      t1, table1, dinv, b1r, W2)

    t2 = _sc_edge_agg(table2, src, dst, ew, _H)
    s_pad, hfin = _tc_call(
        _stage_d_body,
        (jax.ShapeDtypeStruct((_N, 2 * _K), jnp.float32),
         jax.ShapeDtypeStruct((_N, _H), jnp.float32)),
        t2, table2, dinv, b2r, Ws, bsr)

    As = _sc_edge_agg(s_pad, src, dst, ew, 128)
    out, oa, sl, ol, cl = _tc_call(
        _stage_f_body,
        (jax.ShapeDtypeStruct((_K, _H), jnp.float32),
         jax.ShapeDtypeStruct((_K, _K), jnp.float32),
         jax.ShapeDtypeStruct((1, 1), jnp.float32),
         jax.ShapeDtypeStruct((1, 1), jnp.float32),
         jax.ShapeDtypeStruct((1, 1), jnp.float32)),
        s_pad, hfin, As, degp)

    return (s_pad[:, 0:_K], out, oa, sl[0, 0], ol[0, 0], cl[0, 0])


# speedup vs baseline: 11.1397x; 2.5124x over previous
import dataclasses
import functools

import jax
import jax.numpy as jnp
from jax import lax
from jax.experimental import pallas as pl
from jax.experimental.pallas import tpu as pltpu
from jax.experimental.pallas import tpu_sc as plsc

_N = 10000
_E = 320000
_F = 128
_H = 128
_K = 64
_EPS = 1e-15
_SELU_SCALE = 1.0507009873554805
_SELU_ALPHA = 1.6732632423543772


def _selu(v):
    return _SELU_SCALE * jnp.where(v > 0, v, _SELU_ALPHA * (jnp.exp(v) - 1.0))


def _mm(a, b):
    return lax.dot_general(a, b, (((1,), (0,)), ((), ())),
                           preferred_element_type=jnp.float32,
                           precision=lax.Precision.DEFAULT)


def _ct(a, b):
    return lax.dot_general(a, b, (((0,), (0,)), ((), ())),
                           preferred_element_type=jnp.float32,
                           precision=lax.Precision.DEFAULT)


def _stage_a_body(x_ref, w1_ref, h1_ref):
    h1_ref[...] = _mm(x_ref[...], w1_ref[...])


def _stage_b_body(h1_ref, dp_ref, table1_ref, dinv_ref, degp_ref):
    degp = dp_ref[0, :, 0:1] + dp_ref[1, :, 0:1]
    degp_ref[...] = degp
    deg = degp + 1.0
    dinv = lax.rsqrt(jnp.maximum(deg, 1e-12))
    dinv_ref[...] = dinv
    table1_ref[...] = h1_ref[...] * dinv


def _stage_c_body(p_ref, table1_ref, dinv_ref, b1_ref, w2_ref,
                  table2_ref):
    dinv = dinv_ref[...]
    agg = dinv * (p_ref[0] + p_ref[1] + table1_ref[...]) + b1_ref[...]
    z = _selu(agg)
    table2_ref[...] = _mm(z, w2_ref[...]) * dinv


def _stage_d_body(p_ref, table2_ref, dinv_ref, b2_ref, ws_ref,
                  bs_ref, s_ref, hfin_ref):
    dinv = dinv_ref[...]
    agg = dinv * (p_ref[0] + p_ref[1] + table2_ref[...]) + b2_ref[...]
    z = _selu(agg)
    hfin_ref[...] = z
    logits = _mm(z, ws_ref[...]) + bs_ref[...]
    m = jnp.max(logits, axis=-1, keepdims=True)
    e = jnp.exp(logits - m)
    sm = e / jnp.sum(e, axis=-1, keepdims=True)
    s = sm / jnp.maximum(jnp.sum(sm, axis=-1, keepdims=True), _EPS)
    s_ref[:, 0:_K] = s
    s_ref[:, _K:2 * _K] = jnp.zeros((_N, _K), jnp.float32)


def _stage_f_body(s_ref, hfin_ref, a_ref, degp_ref,
                  out_ref, oa_ref, sl_ref, ol_ref, cl_ref):
    s = s_ref[:, 0:_K]
    hfin = hfin_ref[...]
    As = a_ref[0, :, 0:_K] + a_ref[1, :, 0:_K]
    degp = degp_ref[...]

    out_ref[...] = _selu(_ct(s, hfin))
    out_adj = _ct(s, As)
    ca = _ct(s, degp)
    two_m = jnp.maximum(jnp.sum(degp), _EPS)

    eye = (lax.broadcasted_iota(jnp.int32, (_K, _K), 0)
           == lax.broadcasted_iota(jnp.int32, (_K, _K), 1)).astype(jnp.float32)

    tr_adj = jnp.sum(out_adj * eye)
    tr_norm = jnp.sum(ca * ca) / two_m
    sl_ref[...] = jnp.broadcast_to(-(tr_adj - tr_norm) / two_m, (1, 1))

    ss = _ct(s, s)
    nrm_ss = jnp.sqrt(jnp.sum(ss * ss))
    i_nrm = jnp.sqrt(jnp.float32(_K))
    diff = ss / nrm_ss - eye / i_nrm
    ol_ref[...] = jnp.broadcast_to(jnp.sqrt(jnp.sum(diff * diff)), (1, 1))

    cs = jnp.sum(s, axis=0, keepdims=True)
    cl_ref[...] = jnp.broadcast_to(
        jnp.sqrt(jnp.sum(cs * cs)) / _N * jnp.sqrt(jnp.float32(_K)) - 1.0,
        (1, 1))

    m = out_adj * (1.0 - eye)
    d = jnp.sqrt(jnp.maximum(jnp.sum(m, axis=1, keepdims=True), 0.0)) + _EPS
    oa_ref[...] = m / d / jnp.transpose(d)


def _tc_call(body, out_shapes, *args):
    return pl.pallas_call(
        body,
        out_shape=out_shapes,
    )(*args)


_NC = 2
_NS = 16
_NW = _NC * _NS
_EPW = _E // _NW
_BE = 80
_NB = _EPW // _BE
_OSTEP = 624
_OLEN = 640
_ZCH = 128


def _sc_compiler_params():
    cp = pltpu.CompilerParams()
    if "needs_layout_passes" in pltpu.CompilerParams.__dataclass_fields__:
        cp = dataclasses.replace(cp, needs_layout_passes=False)
    return cp


def _sc_edge_agg(table, src, dst, ew, D):
    mesh = plsc.VectorSubcoreMesh(core_axis_name="c", subcore_axis_name="s")
    R = 3

    @functools.partial(
        pl.kernel, mesh=mesh,
        compiler_params=_sc_compiler_params(),
        out_type=jax.ShapeDtypeStruct((_NC, _N, D), jnp.float32),
        scratch_types=(
            [pltpu.VMEM((_BE,), jnp.int32) for _ in range(R)]
            + [pltpu.VMEM((_BE,), jnp.int32) for _ in range(R)]
            + [pltpu.VMEM((_BE,), jnp.float32) for _ in range(R)]
            + [pltpu.VMEM((_BE, 128), jnp.float32) for _ in range(R)]
            + [pltpu.VMEM_SHARED((_N, D), jnp.float32)]
            + [pltpu.SemaphoreType.DMA for _ in range(4 * R)]
        ))
    def k(table_hbm, src_hbm, dst_hbm, ew_hbm, out_hbm, *sc):
        sidx = sc[0:R]
        didx = sc[R:2 * R]
        ewv = sc[2 * R:3 * R]
        grows = sc[3 * R:4 * R]
        acc = sc[4 * R]
        isem = sc[4 * R + 1:4 * R + 1 + R]
        dsem = sc[4 * R + 1 + R:4 * R + 1 + 2 * R]
        wsem = sc[4 * R + 1 + 2 * R:4 * R + 1 + 3 * R]
        gsem = sc[4 * R + 1 + 3 * R:4 * R + 1 + 4 * R]

        cid = lax.axis_index("c")
        sid = lax.axis_index("s")
        w = sid * _NC + cid
        r0 = sid * _OSTEP
        base0 = w * _EPW

        @pl.loop(0, _BE)
        def _(i):
            for j in range(D // 16):
                grows[0][i, pl.ds(j * 16, 16)] = jnp.zeros((16,), jnp.float32)

        @pl.loop(0, _OLEN // _BE)
        def _(cz):
            pltpu.sync_copy(grows[0].at[:, pl.ds(0, D)],
                            acc.at[pl.ds(r0 + cz * _BE, _BE)])

        plsc.subcore_barrier()

        def pf1(b, r):
            pltpu.async_copy(src_hbm.at[pl.ds(base0 + b * _BE, _BE)],
                             sidx[r], isem[r])
            pltpu.async_copy(dst_hbm.at[pl.ds(base0 + b * _BE, _BE)],
                             didx[r], dsem[r])
            pltpu.async_copy(ew_hbm.at[pl.ds(base0 + b * _BE, _BE)],
                             ewv[r], wsem[r])

        def wait_idx(r):
            pltpu.make_async_copy(src_hbm.at[pl.ds(base0, _BE)],
                                  sidx[r], isem[r]).wait()

        def issue_gather(r):
            pltpu.async_copy(table_hbm.at[sidx[r]], grows[r], gsem[r])

        def body(b, r, steady):
            def gather_ahead():
                wait_idx((r + 2) % R)
                issue_gather((r + 2) % R)
            if steady:
                @pl.when(b < _NB - 2)
                def _():
                    gather_ahead()
            pltpu.make_async_copy(table_hbm.at[sidx[r]], grows[r],
                                  gsem[r]).wait()
            pltpu.make_async_copy(dst_hbm.at[pl.ds(base0, _BE)],
                                  didx[r], dsem[r]).wait()
            pltpu.make_async_copy(ew_hbm.at[pl.ds(base0, _BE)],
                                  ewv[r], wsem[r]).wait()

            @pl.loop(0, _BE)
            def _(i):
                wv = plsc.load_gather(ewv[r], [jnp.full((16,), i, jnp.int32)])
                for j in range(D // 16):
                    grows[r][i, pl.ds(j * 16, 16)] = \
                        grows[r][i, pl.ds(j * 16, 16)] * wv

            pltpu.sync_copy(grows[r].at[:, pl.ds(0, D)], acc.at[didx[r]],
                            add=True)
            if steady:
                @pl.when(b < _NB - 3)
                def _():
                    pf1(b + 3, r)

        for r in range(3):
            pf1(r, r)
        for r in range(2):
            wait_idx(r)
            issue_gather(r)

        @pl.loop(0, (_NB - 2) // R)
        def _(g):
            for r in range(R):
                body(g * R + r, r, True)

        for b in range(_NB - 2, _NB):
            body(b, b % R, False)

        plsc.subcore_barrier()
        pltpu.sync_copy(acc.at[pl.ds(r0, _OLEN)],
                        out_hbm.at[cid, pl.ds(r0, _OLEN)])

    return k(table, src, dst, ew)


def _sc_deg(src, dst, ew):
    mesh = plsc.VectorSubcoreMesh(core_axis_name="c", subcore_axis_name="s")

    @functools.partial(
        pl.kernel, mesh=mesh,
        compiler_params=_sc_compiler_params(),
        out_type=jax.ShapeDtypeStruct((_NC, _N, 128), jnp.float32),
        scratch_types=[
            pltpu.VMEM((_BE,), jnp.int32),
            pltpu.VMEM((_BE,), jnp.float32),
            pltpu.VMEM((_BE, 128), jnp.float32),
            pltpu.VMEM((_ZCH, 128), jnp.float32),
            pltpu.VMEM_SHARED((_N, 128), jnp.float32),
        ])
    def k(src_hbm, dst_hbm, ew_hbm, out_hbm, didx, ewv, rows, zbuf, acc):
        cid = lax.axis_index("c")
        sid = lax.axis_index("s")
        w = sid * _NC + cid
        r0 = sid * _OSTEP

        @pl.loop(0, _ZCH)
        def _(i):
            for j in range(8):
                zbuf[i, pl.ds(j * 16, 16)] = jnp.zeros((16,), jnp.float32)

        @pl.loop(0, _OLEN // _ZCH)
        def _(cz):
            pltpu.sync_copy(zbuf, acc.at[pl.ds(r0 + cz * _ZCH, _ZCH)])

        plsc.subcore_barrier()

        base0 = w * _EPW

        @pl.loop(0, _NB)
        def _(b):
            base = base0 + b * _BE
            pltpu.sync_copy(ew_hbm.at[pl.ds(base, _BE)], ewv)
            pltpu.sync_copy(dst_hbm.at[pl.ds(base, _BE)], didx)

            @pl.loop(0, _BE)
            def _(i):
                wv = plsc.load_gather(ewv, [jnp.full((16,), i, jnp.int32)])
                for j in range(8):
                    rows[i, pl.ds(j * 16, 16)] = wv

            pltpu.sync_copy(rows, acc.at[didx], add=True)

        plsc.subcore_barrier()
        pltpu.sync_copy(acc.at[pl.ds(r0, _OLEN)],
                        out_hbm.at[cid, pl.ds(r0, _OLEN)])

    return k(src, dst, ew)


def kernel(x, edge_index, edge_weight, W1, b1, W2, b2, Ws, bs):
    src, dst = edge_index[0], edge_index[1]
    ew = edge_weight
    b1r = b1.reshape(1, _H)
    b2r = b2.reshape(1, _H)
    bsr = bs.reshape(1, _K)

    h1 = _tc_call(_stage_a_body, jax.ShapeDtypeStruct((_N, _H), jnp.float32),
                  x, W1)
    degP = _sc_deg(src, dst, ew)

    table1, dinv, degp = _tc_call(
        _stage_b_body,
        (jax.ShapeDtypeStruct((_N, _H), jnp.float32),
         jax.ShapeDtypeStruct((_N, 1), jnp.float32),
         jax.ShapeDtypeStruct((_N, 1), jnp.float32)),
        h1, degP)

    t1 = _sc_edge_agg(table1, src, dst, ew, _H)
    table2 = _tc_call(_stage_c_body,
                      jax.ShapeDtypeStruct((_N, _H), jnp.float32),
                      t1, table1, dinv, b1r, W2)

    t2 = _sc_edge_agg(table2, src, dst, ew, _H)
    s_pad, hfin = _tc_call(
        _stage_d_body,
        (jax.ShapeDtypeStruct((_N, 2 * _K), jnp.float32),
         jax.ShapeDtypeStruct((_N, _H), jnp.float32)),
        t2, table2, dinv, b2r, Ws, bsr)

    As = _sc_edge_agg(s_pad, src, dst, ew, 128)
    out, oa, sl, ol, cl = _tc_call(
        _stage_f_body,
        (jax.ShapeDtypeStruct((_K, _H), jnp.float32),
         jax.ShapeDtypeStruct((_K, _K), jnp.float32),
         jax.ShapeDtypeStruct((1, 1), jnp.float32),
         jax.ShapeDtypeStruct((1, 1), jnp.float32),
         jax.ShapeDtypeStruct((1, 1), jnp.float32)),
        s_pad, hfin, As, degp)

    return (s_pad[:, 0:_K], out, oa, sl[0, 0], ol[0, 0], cl[0, 0])


# speedup vs baseline: 16.1042x; 1.4457x over previous
import dataclasses
import functools

import jax
import jax.numpy as jnp
from jax import lax
from jax.experimental import pallas as pl
from jax.experimental.pallas import tpu as pltpu
from jax.experimental.pallas import tpu_sc as plsc

_N = 10000
_E = 320000
_F = 128
_H = 128
_K = 64
_EPS = 1e-15
_SELU_SCALE = 1.0507009873554805
_SELU_ALPHA = 1.6732632423543772


def _selu(v):
    return _SELU_SCALE * jnp.where(v > 0, v, _SELU_ALPHA * (jnp.exp(v) - 1.0))


def _mm(a, b):
    return lax.dot_general(a, b, (((1,), (0,)), ((), ())),
                           preferred_element_type=jnp.float32,
                           precision=lax.Precision.DEFAULT)


def _ct(a, b):
    return lax.dot_general(a, b, (((0,), (0,)), ((), ())),
                           preferred_element_type=jnp.float32,
                           precision=lax.Precision.DEFAULT)


def _stage_a_body(x_ref, w1_ref, h1_ref):
    h1_ref[...] = _mm(x_ref[...], w1_ref[...])


def _stage_b_body(h1_ref, dp_ref, table1_ref, dinv_ref, degp_ref):
    degp = dp_ref[0, :, 0:1] + dp_ref[1, :, 0:1]
    degp_ref[...] = degp
    deg = degp + 1.0
    dinv = lax.rsqrt(jnp.maximum(deg, 1e-12))
    dinv_ref[...] = dinv
    table1_ref[...] = h1_ref[...] * dinv


def _stage_c_body(p_ref, table1_ref, dinv_ref, b1_ref, w2_ref,
                  table2_ref):
    dinv = dinv_ref[...]
    agg = dinv * (p_ref[0] + p_ref[1] + table1_ref[...]) + b1_ref[...]
    z = _selu(agg)
    table2_ref[...] = _mm(z, w2_ref[...]) * dinv


def _stage_d_body(p_ref, table2_ref, dinv_ref, b2_ref, ws_ref,
                  bs_ref, s_ref, hfin_ref):
    dinv = dinv_ref[...]
    agg = dinv * (p_ref[0] + p_ref[1] + table2_ref[...]) + b2_ref[...]
    z = _selu(agg)
    hfin_ref[...] = z
    logits = _mm(z, ws_ref[...]) + bs_ref[...]
    m = jnp.max(logits, axis=-1, keepdims=True)
    e = jnp.exp(logits - m)
    sm = e / jnp.sum(e, axis=-1, keepdims=True)
    s = sm / jnp.maximum(jnp.sum(sm, axis=-1, keepdims=True), _EPS)
    s_ref[:, 0:_K] = s
    s_ref[:, _K:2 * _K] = jnp.zeros((_N, _K), jnp.float32)


def _stage_f_body(s_ref, hfin_ref, a_ref, degp_ref,
                  out_ref, oa_ref, sl_ref, ol_ref, cl_ref):
    s = s_ref[:, 0:_K]
    hfin = hfin_ref[...]
    As = a_ref[0, :, 0:_K] + a_ref[1, :, 0:_K]
    degp = degp_ref[...]

    out_ref[...] = _selu(_ct(s, hfin))
    out_adj = _ct(s, As)
    ca = _ct(s, degp)
    two_m = jnp.maximum(jnp.sum(degp), _EPS)

    eye = (lax.broadcasted_iota(jnp.int32, (_K, _K), 0)
           == lax.broadcasted_iota(jnp.int32, (_K, _K), 1)).astype(jnp.float32)

    tr_adj = jnp.sum(out_adj * eye)
    tr_norm = jnp.sum(ca * ca) / two_m
    sl_ref[...] = jnp.broadcast_to(-(tr_adj - tr_norm) / two_m, (1, 1))

    ss = _ct(s, s)
    nrm_ss = jnp.sqrt(jnp.sum(ss * ss))
    i_nrm = jnp.sqrt(jnp.float32(_K))
    diff = ss / nrm_ss - eye / i_nrm
    ol_ref[...] = jnp.broadcast_to(jnp.sqrt(jnp.sum(diff * diff)), (1, 1))

    cs = jnp.sum(s, axis=0, keepdims=True)
    cl_ref[...] = jnp.broadcast_to(
        jnp.sqrt(jnp.sum(cs * cs)) / _N * jnp.sqrt(jnp.float32(_K)) - 1.0,
        (1, 1))

    m = out_adj * (1.0 - eye)
    d = jnp.sqrt(jnp.maximum(jnp.sum(m, axis=1, keepdims=True), 0.0)) + _EPS
    oa_ref[...] = m / d / jnp.transpose(d)


def _tc_call(body, out_shapes, *args):
    return pl.pallas_call(
        body,
        out_shape=out_shapes,
    )(*args)


_NC = 2
_NS = 16
_NW = _NC * _NS
_EPW = _E // _NW
_BE = 80
_NB = _EPW // _BE
_OSTEP = 624
_OLEN = 640
_ZCH = 128


def _sc_compiler_params():
    cp = pltpu.CompilerParams()
    if "needs_layout_passes" in pltpu.CompilerParams.__dataclass_fields__:
        cp = dataclasses.replace(cp, needs_layout_passes=False)
    return cp


def _sc_pass(src, dst, ew, table, D):
    mesh = plsc.VectorSubcoreMesh(core_axis_name="c", subcore_axis_name="s")
    R = 3
    has_tab = table is not None

    @functools.partial(
        pl.kernel, mesh=mesh,
        compiler_params=_sc_compiler_params(),
        out_type=jax.ShapeDtypeStruct((_NC, _N, D), jnp.float32),
        scratch_types=(
            [pltpu.VMEM((_BE,), jnp.int32) for _ in range(R)]
            + [pltpu.VMEM((_BE,), jnp.int32) for _ in range(R)]
            + [pltpu.VMEM((_BE,), jnp.float32) for _ in range(R)]
            + [pltpu.VMEM((_BE, 128), jnp.float32) for _ in range(R)]
            + [pltpu.VMEM_SHARED((_N, D), jnp.float32)]
            + [pltpu.SemaphoreType.DMA for _ in range(5 * R)]
        ))
    def k(*refs):
        if has_tab:
            table_hbm, src_hbm, dst_hbm, ew_hbm, out_hbm = refs[:5]
            sc = refs[5:]
        else:
            src_hbm, dst_hbm, ew_hbm, out_hbm = refs[:4]
            sc = refs[4:]
        sidx = sc[0:R]
        didx = sc[R:2 * R]
        ewv = sc[2 * R:3 * R]
        grows = sc[3 * R:4 * R]
        acc = sc[4 * R]
        isem = sc[4 * R + 1:4 * R + 1 + R]
        dsem = sc[4 * R + 1 + R:4 * R + 1 + 2 * R]
        wsem = sc[4 * R + 1 + 2 * R:4 * R + 1 + 3 * R]
        gsem = sc[4 * R + 1 + 3 * R:4 * R + 1 + 4 * R]
        ssem = sc[4 * R + 1 + 4 * R:4 * R + 1 + 5 * R]

        cid = lax.axis_index("c")
        sid = lax.axis_index("s")
        w = sid * _NC + cid
        r0 = sid * _OSTEP
        base0 = w * _EPW

        @pl.loop(0, _BE)
        def _(i):
            for j in range(D // 16):
                grows[0][i, pl.ds(j * 16, 16)] = jnp.zeros((16,), jnp.float32)

        @pl.loop(0, _OLEN // _BE)
        def _(cz):
            pltpu.sync_copy(grows[0].at[:, pl.ds(0, D)],
                            acc.at[pl.ds(r0 + cz * _BE, _BE)])

        plsc.subcore_barrier()

        def pf1(b, r):
            if has_tab:
                pltpu.async_copy(src_hbm.at[pl.ds(base0 + b * _BE, _BE)],
                                 sidx[r], isem[r])
            pltpu.async_copy(dst_hbm.at[pl.ds(base0 + b * _BE, _BE)],
                             didx[r], dsem[r])
            pltpu.async_copy(ew_hbm.at[pl.ds(base0 + b * _BE, _BE)],
                             ewv[r], wsem[r])

        def wait_idx(r):
            pltpu.make_async_copy(src_hbm.at[pl.ds(base0, _BE)],
                                  sidx[r], isem[r]).wait()

        def issue_gather(r):
            pltpu.async_copy(table_hbm.at[sidx[r]], grows[r], gsem[r])

        def wait_scat(r):
            pltpu.make_async_copy(grows[r].at[:, pl.ds(0, D)],
                                  acc.at[didx[r]], ssem[r]).wait()

        def body(b, r, steady):
            if has_tab:
                pltpu.make_async_copy(table_hbm.at[sidx[r]], grows[r],
                                      gsem[r]).wait()
            pltpu.make_async_copy(dst_hbm.at[pl.ds(base0, _BE)],
                                  didx[r], dsem[r]).wait()
            pltpu.make_async_copy(ew_hbm.at[pl.ds(base0, _BE)],
                                  ewv[r], wsem[r]).wait()

            @pl.loop(0, _BE)
            def _(i):
                wv = plsc.load_gather(ewv[r], [jnp.full((16,), i, jnp.int32)])
                for j in range(D // 16):
                    if has_tab:
                        grows[r][i, pl.ds(j * 16, 16)] = \
                            grows[r][i, pl.ds(j * 16, 16)] * wv
                    else:
                        grows[r][i, pl.ds(j * 16, 16)] = wv

            rr = (r + 2) % R

            @pl.when(b >= 1)
            def _():
                wait_scat(rr)

            if steady:
                @pl.when(b < _NB - 2)
                def _():
                    pf1(b + 2, rr)
                    if has_tab:
                        wait_idx(rr)
                        issue_gather(rr)

            pltpu.async_copy(grows[r].at[:, pl.ds(0, D)], acc.at[didx[r]],
                             ssem[r], add=True)

        for r in range(2):
            pf1(r, r)
        if has_tab:
            for r in range(2):
                wait_idx(r)
                issue_gather(r)

        @pl.loop(0, (_NB - 2) // R)
        def _(g):
            for r in range(R):
                body(g * R + r, r, True)

        for b in range(_NB - 2, _NB):
            body(b, b % R, False)

        wait_scat((_NB - 1) % R)

        plsc.subcore_barrier()
        pltpu.sync_copy(acc.at[pl.ds(r0, _OLEN)],
                        out_hbm.at[cid, pl.ds(r0, _OLEN)])

    if has_tab:
        return k(table, src, dst, ew)
    return k(src, dst, ew)


def _sc_edge_agg(table, src, dst, ew, D):
    return _sc_pass(src, dst, ew, table, D)


def _sc_deg(src, dst, ew):
    return _sc_pass(src, dst, ew, None, 128)


def kernel(x, edge_index, edge_weight, W1, b1, W2, b2, Ws, bs):
    src, dst = edge_index[0], edge_index[1]
    ew = edge_weight
    b1r = b1.reshape(1, _H)
    b2r = b2.reshape(1, _H)
    bsr = bs.reshape(1, _K)

    h1 = _tc_call(_stage_a_body, jax.ShapeDtypeStruct((_N, _H), jnp.float32),
                  x, W1)
    degP = _sc_deg(src, dst, ew)

    table1, dinv, degp = _tc_call(
        _stage_b_body,
        (jax.ShapeDtypeStruct((_N, _H), jnp.float32),
         jax.ShapeDtypeStruct((_N, 1), jnp.float32),
         jax.ShapeDtypeStruct((_N, 1), jnp.float32)),
        h1, degP)

    t1 = _sc_edge_agg(table1, src, dst, ew, _H)
    table2 = _tc_call(_stage_c_body,
                      jax.ShapeDtypeStruct((_N, _H), jnp.float32),
                      t1, table1, dinv, b1r, W2)

    t2 = _sc_edge_agg(table2, src, dst, ew, _H)
    s_pad, hfin = _tc_call(
        _stage_d_body,
        (jax.ShapeDtypeStruct((_N, 2 * _K), jnp.float32),
         jax.ShapeDtypeStruct((_N, _H), jnp.float32)),
        t2, table2, dinv, b2r, Ws, bsr)

    As = _sc_edge_agg(s_pad, src, dst, ew, 128)
    out, oa, sl, ol, cl = _tc_call(
        _stage_f_body,
        (jax.ShapeDtypeStruct((_K, _H), jnp.float32),
         jax.ShapeDtypeStruct((_K, _K), jnp.float32),
         jax.ShapeDtypeStruct((1, 1), jnp.float32),
         jax.ShapeDtypeStruct((1, 1), jnp.float32),
         jax.ShapeDtypeStruct((1, 1), jnp.float32)),
        s_pad, hfin, As, degp)

    return (s_pad[:, 0:_K], out, oa, sl[0, 0], ol[0, 0], cl[0, 0])
